# baseline (speedup 1.0000x reference)
import jax
import jax.numpy as jnp
from jax import lax
from jax.experimental import pallas as pl
from jax.experimental.pallas import tpu as pltpu

N_NODES = 100000
D_FEAT = 256
NUM_GRAPHS = 512

BH = 2000
NH = 5
BM = BH * NH
W = 64
NB = N_NODES // BM

NEG_INF = float("-inf")


def _attn_pool_kernel(first_ref, last_ref, x_ref, i_ref, a_ref, out_ref,
                      stat_ref):
    k = pl.program_id(0)

    @pl.when(k == 0)
    def _init():
        out_ref[...] = jnp.zeros_like(out_ref)
        stat_ref[0] = jnp.float32(NEG_INF)
        stat_ref[1] = jnp.float32(0.0)

    a = a_ref[...].astype(jnp.bfloat16)

    xs = []
    cs = []
    for h in range(NH):
        x = x_ref[h * BH:(h + 1) * BH, :].astype(jnp.bfloat16)
        xs.append(x)
        cs.append(lax.dot_general(a, x, (((0,), (1,)), ((), ())),
                                  preferred_element_type=jnp.float32))

    iota = lax.broadcasted_iota(jnp.int32, (W, BH), 0)
    first = first_ref[NH * k]
    base0 = pl.multiple_of(jnp.minimum((first // 8) * 8, NUM_GRAPHS - W), 8)
    hits = []
    for h in range(NH):
        i_row = i_ref[0, :, h * BH:(h + 1) * BH]
        hits.append((iota + base0) == i_row)

    m_old = stat_ref[0]
    m_blk = jnp.float32(NEG_INF)
    for c in cs:
        m_blk = jnp.maximum(m_blk, jnp.max(c))
    m_new = jnp.maximum(m_old, m_blk)
    alpha = jnp.exp(m_old - m_new)

    ps = [jnp.exp(c - m_new) for c in cs]
    s = jnp.float32(0.0)
    for p in ps:
        s = s + jnp.sum(p)
    stat_ref[0] = m_new
    stat_ref[1] = stat_ref[1] * alpha + s

    @pl.when(m_blk > m_old)
    def _rescale():
        out_ref[...] = out_ref[...] * alpha

    acc = None
    for h in range(NH):
        ohp0 = jnp.where(hits[h], ps[h], jnp.float32(0.0)).astype(jnp.bfloat16)
        contrib = jnp.dot(ohp0, xs[h], preferred_element_type=jnp.float32)
        acc = contrib if acc is None else acc + contrib

    out_ref[pl.ds(base0, W), :] += acc

    @pl.when(last_ref[NH * k + NH - 1] >= base0 + W)
    def _fallback():
        for h in range(NH):
            x = xs[h]
            p = ps[h]
            i_row = i_ref[0, :, h * BH:(h + 1) * BH]
            last = last_ref[NH * k + h]

            def more(l, p=p, x=x, i_row=i_row):
                base = pl.multiple_of(jnp.minimum(l, NUM_GRAPHS - W), 8)
                hit = ((iota + base) == i_row) & (i_row >= l)
                ohp = jnp.where(hit, p, jnp.float32(0.0)).astype(jnp.bfloat16)
                contrib = jnp.dot(ohp, x, preferred_element_type=jnp.float32)
                out_ref[pl.ds(base, W), :] += contrib
                return base + W

            lax.while_loop(lambda l: l <= last, more, base0 + W)

    @pl.when(k == NB - 1)
    def _finalize():
        out_ref[...] = out_ref[...] / stat_ref[1]


def kernel(X, I, attn_kernel):
    I32 = I.astype(jnp.int32)
    first = I32[0::BH]
    last = I32[BH - 1::BH]
    I3 = I32.reshape(NB, 1, BM)
    grid_spec = pltpu.PrefetchScalarGridSpec(
        num_scalar_prefetch=2,
        grid=(NB,),
        in_specs=[
            pl.BlockSpec((BM, D_FEAT), lambda i, f, l: (i, 0)),
            pl.BlockSpec((1, 1, BM), lambda i, f, l: (i, 0, 0)),
            pl.BlockSpec((D_FEAT, 1), lambda i, f, l: (0, 0)),
        ],
        out_specs=pl.BlockSpec((NUM_GRAPHS, D_FEAT), lambda i, f, l: (0, 0)),
        scratch_shapes=[pltpu.SMEM((2,), jnp.float32)],
    )
    return pl.pallas_call(
        _attn_pool_kernel,
        grid_spec=grid_spec,
        out_shape=jax.ShapeDtypeStruct((NUM_GRAPHS, D_FEAT), jnp.float32),
        compiler_params=pltpu.CompilerParams(
            dimension_semantics=("arbitrary",),
        ),
    )(first, last, X, I3, attn_kernel)


# speedup vs baseline: 1.2054x; 1.2054x over previous
import jax
import jax.numpy as jnp
from jax import lax
from jax.experimental import pallas as pl
from jax.experimental.pallas import tpu as pltpu

N_NODES = 100000
D_FEAT = 256
NUM_GRAPHS = 512

BH = 2000
NH = 10
BM = BH * NH
W = 32
NB = N_NODES // BM

NEG_INF = float("-inf")


def _attn_pool_kernel(first_ref, last_ref, x_ref, i_ref, a_ref, out_ref,
                      stat_ref):
    k = pl.program_id(0)

    @pl.when(k == 0)
    def _init():
        out_ref[...] = jnp.zeros_like(out_ref)
        stat_ref[0] = jnp.float32(NEG_INF)
        stat_ref[1] = jnp.float32(0.0)

    a = a_ref[...].astype(jnp.bfloat16)

    xs = []
    cs = []
    for h in range(NH):
        x = x_ref[h * BH:(h + 1) * BH, :].astype(jnp.bfloat16)
        xs.append(x)
        cs.append(lax.dot_general(a, x, (((0,), (1,)), ((), ())),
                                  preferred_element_type=jnp.float32))

    iota = lax.broadcasted_iota(jnp.int32, (W, BH), 0)
    bases = []
    hits = []
    for h in range(NH):
        i_row = i_ref[0, :, h * BH:(h + 1) * BH]
        first = first_ref[NH * k + h]
        base0 = pl.multiple_of(
            jnp.minimum((first // 8) * 8, NUM_GRAPHS - W), 8)
        bases.append(base0)
        hits.append((iota + base0) == i_row)

    m_old = stat_ref[0]
    m_blk = jnp.float32(NEG_INF)
    for c in cs:
        m_blk = jnp.maximum(m_blk, jnp.max(c))
    m_new = jnp.maximum(m_old, m_blk)
    alpha = jnp.exp(m_old - m_new)

    ps = [jnp.exp(c - m_new) for c in cs]
    s = jnp.float32(0.0)
    for p in ps:
        s = s + jnp.sum(p)
    stat_ref[0] = m_new
    stat_ref[1] = stat_ref[1] * alpha + s

    @pl.when(m_blk > m_old)
    def _rescale():
        out_ref[...] = out_ref[...] * alpha

    contribs = []
    for h in range(NH):
        ohp0 = jnp.where(hits[h], ps[h], jnp.float32(0.0)).astype(jnp.bfloat16)
        contribs.append(
            jnp.dot(ohp0, xs[h], preferred_element_type=jnp.float32))

    for h in range(NH):
        out_ref[pl.ds(bases[h], W), :] += contribs[h]

    need_more = last_ref[NH * k] >= bases[0] + W
    for h in range(1, NH):
        need_more |= last_ref[NH * k + h] >= bases[h] + W

    @pl.when(need_more)
    def _fallback():
        for h in range(NH):
            x = xs[h]
            p = ps[h]
            i_row = i_ref[0, :, h * BH:(h + 1) * BH]
            last = last_ref[NH * k + h]

            def more(l, p=p, x=x, i_row=i_row):
                base = pl.multiple_of(jnp.minimum(l, NUM_GRAPHS - W), 8)
                hit = ((iota + base) == i_row) & (i_row >= l)
                ohp = jnp.where(hit, p, jnp.float32(0.0)).astype(jnp.bfloat16)
                contrib = jnp.dot(ohp, x, preferred_element_type=jnp.float32)
                out_ref[pl.ds(base, W), :] += contrib
                return base + W

            lax.while_loop(lambda l: l <= last, more, bases[h] + W)

    @pl.when(k == NB - 1)
    def _finalize():
        out_ref[...] = out_ref[...] / stat_ref[1]


def kernel(X, I, attn_kernel):
    I32 = I.astype(jnp.int32)
    first = I32[0::BH]
    last = I32[BH - 1::BH]
    I3 = I32.reshape(NB, 1, BM)
    grid_spec = pltpu.PrefetchScalarGridSpec(
        num_scalar_prefetch=2,
        grid=(NB,),
        in_specs=[
            pl.BlockSpec((BM, D_FEAT), lambda i, f, l: (i, 0)),
            pl.BlockSpec((1, 1, BM), lambda i, f, l: (i, 0, 0)),
            pl.BlockSpec((D_FEAT, 1), lambda i, f, l: (0, 0)),
        ],
        out_specs=pl.BlockSpec((NUM_GRAPHS, D_FEAT), lambda i, f, l: (0, 0)),
        scratch_shapes=[pltpu.SMEM((2,), jnp.float32)],
    )
    return pl.pallas_call(
        _attn_pool_kernel,
        grid_spec=grid_spec,
        out_shape=jax.ShapeDtypeStruct((NUM_GRAPHS, D_FEAT), jnp.float32),
        compiler_params=pltpu.CompilerParams(
            dimension_semantics=("arbitrary",),
        ),
    )(first, last, X, I3, attn_kernel)


# speedup vs baseline: 1.2064x; 1.0009x over previous
import jax
import jax.numpy as jnp
from jax import lax
from jax.experimental import pallas as pl
from jax.experimental.pallas import tpu as pltpu

N_NODES = 100000
D_FEAT = 256
NUM_GRAPHS = 512

BH = 2000
NH = 10
BM = BH * NH
W = 32
NB = N_NODES // BM

NEG_INF = float("-inf")


def _attn_pool_kernel(first_ref, last_ref, x_ref, i_ref, a_ref, out_ref,
                      stat_ref, pend_ref):
    k = pl.program_id(0)

    @pl.when(k == 0)
    def _init():
        out_ref[...] = jnp.zeros_like(out_ref)
        pend_ref[...] = jnp.zeros_like(pend_ref)
        stat_ref[0] = jnp.float32(NEG_INF)
        stat_ref[1] = jnp.float32(0.0)

    a = a_ref[...].astype(jnp.bfloat16)

    for h in range(NH):
        pfirst = first_ref[jnp.maximum(NH * k - NH + h, 0)]
        pbase = pl.multiple_of(
            jnp.minimum((pfirst // 8) * 8, NUM_GRAPHS - W), 8)
        out_ref[pl.ds(pbase, W), :] += pend_ref[h]

    xs = []
    cs = []
    for h in range(NH):
        x = x_ref[h * BH:(h + 1) * BH, :].astype(jnp.bfloat16)
        xs.append(x)
        cs.append(lax.dot_general(a, x, (((0,), (1,)), ((), ())),
                                  preferred_element_type=jnp.float32))

    iota = lax.broadcasted_iota(jnp.int32, (W, BH), 0)
    bases = []
    hits = []
    for h in range(NH):
        i_row = i_ref[0, :, h * BH:(h + 1) * BH]
        first = first_ref[NH * k + h]
        base0 = pl.multiple_of(
            jnp.minimum((first // 8) * 8, NUM_GRAPHS - W), 8)
        bases.append(base0)
        hits.append((iota + base0) == i_row)

    m_old = stat_ref[0]
    m_blk = jnp.float32(NEG_INF)
    for c in cs:
        m_blk = jnp.maximum(m_blk, jnp.max(c))
    m_new = jnp.maximum(m_old, m_blk)
    alpha = jnp.exp(m_old - m_new)

    ps = [jnp.exp(c - m_new) for c in cs]
    s = jnp.float32(0.0)
    for p in ps:
        s = s + jnp.sum(p)
    stat_ref[0] = m_new
    stat_ref[1] = stat_ref[1] * alpha + s

    @pl.when(m_blk > m_old)
    def _rescale():
        out_ref[...] = out_ref[...] * alpha

    contribs = []
    for h in range(NH):
        ohp0 = jnp.where(hits[h], ps[h], jnp.float32(0.0)).astype(jnp.bfloat16)
        contribs.append(
            jnp.dot(ohp0, xs[h], preferred_element_type=jnp.float32))

    for h in range(NH):
        pend_ref[h] = contribs[h]

    need_more = last_ref[NH * k] >= bases[0] + W
    for h in range(1, NH):
        need_more |= last_ref[NH * k + h] >= bases[h] + W

    @pl.when(need_more)
    def _fallback():
        for h in range(NH):
            x = xs[h]
            p = ps[h]
            i_row = i_ref[0, :, h * BH:(h + 1) * BH]
            last = last_ref[NH * k + h]

            def more(l, p=p, x=x, i_row=i_row):
                base = pl.multiple_of(jnp.minimum(l, NUM_GRAPHS - W), 8)
                hit = ((iota + base) == i_row) & (i_row >= l)
                ohp = jnp.where(hit, p, jnp.float32(0.0)).astype(jnp.bfloat16)
                contrib = jnp.dot(ohp, x, preferred_element_type=jnp.float32)
                out_ref[pl.ds(base, W), :] += contrib
                return base + W

            lax.while_loop(lambda l: l <= last, more, bases[h] + W)

    @pl.when(k == NB - 1)
    def _finalize():
        for h in range(NH):
            out_ref[pl.ds(bases[h], W), :] += contribs[h]
        out_ref[...] = out_ref[...] / stat_ref[1]


def kernel(X, I, attn_kernel):
    I32 = I.astype(jnp.int32)
    first = I32[0::BH]
    last = I32[BH - 1::BH]
    I3 = I32.reshape(NB, 1, BM)
    grid_spec = pltpu.PrefetchScalarGridSpec(
        num_scalar_prefetch=2,
        grid=(NB,),
        in_specs=[
            pl.BlockSpec((BM, D_FEAT), lambda i, f, l: (i, 0)),
            pl.BlockSpec((1, 1, BM), lambda i, f, l: (i, 0, 0)),
            pl.BlockSpec((D_FEAT, 1), lambda i, f, l: (0, 0)),
        ],
        out_specs=pl.BlockSpec((NUM_GRAPHS, D_FEAT), lambda i, f, l: (0, 0)),
        scratch_shapes=[pltpu.SMEM((2,), jnp.float32),
                        pltpu.VMEM((NH, W, D_FEAT), jnp.float32)],
    )
    return pl.pallas_call(
        _attn_pool_kernel,
        grid_spec=grid_spec,
        out_shape=jax.ShapeDtypeStruct((NUM_GRAPHS, D_FEAT), jnp.float32),
        compiler_params=pltpu.CompilerParams(
            dimension_semantics=("arbitrary",),
        ),
    )(first, last, X, I3, attn_kernel)
